# Initial kernel scaffold; baseline (speedup 1.0000x reference)
import functools

import jax
import jax.numpy as jnp
from jax import lax
from jax.experimental import pallas as pl
from jax.experimental.pallas import tpu as pltpu
from jax.experimental.pallas import tpu_sc as plsc

E = 8
K = 2
D = 1024
F = 2048
M = 2048

T = 256
NP = M * K + (E - 1) * T
NB = NP // T

NC = 2
NS = 16
NW = NC * NS


def _sc_gather(table, idx, n_chunks):
    R, Dd = table.shape
    B = idx.shape[0]
    b_per_w = B // NW
    ch = b_per_w // n_chunks
    mesh = plsc.VectorSubcoreMesh(
        core_axis_name="c", subcore_axis_name="s", num_cores=NC, num_subcores=NS
    )

    @functools.partial(
        pl.kernel,
        out_type=jax.ShapeDtypeStruct((B, Dd), table.dtype),
        mesh=mesh,
        scratch_types=[
            pltpu.VMEM((n_chunks, ch), jnp.int32),
            pltpu.VMEM((ch, Dd), table.dtype),
            pltpu.SemaphoreType.DMA,
        ],
    )
    def k(table_hbm, idx_hbm, out_hbm, idx_v, rows_v, sem):
        wid = lax.axis_index("s") * NC + lax.axis_index("c")
        base = wid * b_per_w
        pltpu.sync_copy(
            idx_hbm.at[pl.ds(base, b_per_w)],
            idx_v.at[...].reshape(b_per_w),
        )
        for c in range(n_chunks):
            pltpu.async_copy(table_hbm.at[idx_v.at[c]], rows_v, sem).wait()
            pltpu.sync_copy(rows_v, out_hbm.at[pl.ds(base + c * ch, ch)])

    return k(table, idx)


def _tc_gmm(xs, w1, w2, w3, block_expert):
    def body(be_ref, x_ref, w1_ref, w3_ref, w2_ref, o_ref):
        x = x_ref[...]
        h = jnp.dot(x, w1_ref[0], preferred_element_type=jnp.float32)
        g = jnp.dot(x, w3_ref[0], preferred_element_type=jnp.float32)
        a = h * jax.nn.sigmoid(h) * g
        o_ref[...] = jnp.dot(a, w2_ref[0], preferred_element_type=jnp.float32)

    grid_spec = pltpu.PrefetchScalarGridSpec(
        num_scalar_prefetch=1,
        grid=(NB,),
        in_specs=[
            pl.BlockSpec((T, D), lambda b, be: (b, 0)),
            pl.BlockSpec((1, D, F), lambda b, be: (be[b], 0, 0)),
            pl.BlockSpec((1, D, F), lambda b, be: (be[b], 0, 0)),
            pl.BlockSpec((1, F, D), lambda b, be: (be[b], 0, 0)),
        ],
        out_specs=pl.BlockSpec((T, D), lambda b, be: (b, 0)),
    )
    return pl.pallas_call(
        body,
        grid_spec=grid_spec,
        out_shape=jax.ShapeDtypeStruct((NP, D), jnp.float32),
    )(block_expert, xs, w1, w3, w2)


def _route(top_ks):
    top_flat = top_ks.reshape(-1).astype(jnp.int32)
    counts = jnp.zeros((E,), jnp.int32).at[top_flat].add(1)
    padded = ((counts + T - 1) // T) * T
    offs_p = jnp.concatenate(
        [jnp.zeros((1,), jnp.int32), jnp.cumsum(padded)[:-1]]
    )
    offs_u = jnp.concatenate(
        [jnp.zeros((1,), jnp.int32), jnp.cumsum(counts)[:-1]]
    )
    order = jnp.argsort(top_flat, stable=True).astype(jnp.int32)
    eid = top_flat[order]
    dest = offs_p[eid] + (jnp.arange(M * K, dtype=jnp.int32) - offs_u[eid])
    sidx = jnp.zeros((NP,), jnp.int32).at[dest].set(order // K)
    pos = jnp.zeros((M * K,), jnp.int32).at[order].set(dest)
    b_idx = jnp.arange(NB, dtype=jnp.int32)
    be = (
        jnp.sum((b_idx[None, :] >= (offs_p // T)[:, None]).astype(jnp.int32), axis=0)
        - 1
    )
    return sidx, pos, be.astype(jnp.int32)


def kernel(hidden_states, top_ks, w1, w2, w3):
    sidx, pos, be = _route(top_ks)
    xs = _sc_gather(hidden_states, sidx, n_chunks=4)
    ys = _tc_gmm(xs, w1, w2, w3, be)
    out = _sc_gather(ys, pos, n_chunks=2)
    return out.reshape(M, K, D)


# speedup vs baseline: 2.4409x; 2.4409x over previous
import functools

import jax
import jax.numpy as jnp
from jax import lax
from jax.experimental import pallas as pl
from jax.experimental.pallas import tpu as pltpu
from jax.experimental.pallas import tpu_sc as plsc

E = 8
K = 2
D = 1024
F = 2048
M = 2048

T = 256
NP = 6144
NB = NP // T

NC = 2
NS = 16
NW = NC * NS


def _sc_gather(table, idx, n_chunks):
    R, Dd = table.shape
    B = idx.shape[0]
    b_per_w = B // NW
    ch = b_per_w // n_chunks
    mesh = plsc.VectorSubcoreMesh(
        core_axis_name="c", subcore_axis_name="s", num_cores=NC, num_subcores=NS
    )

    @functools.partial(
        pl.kernel,
        out_type=jax.ShapeDtypeStruct((B, Dd), table.dtype),
        mesh=mesh,
        scratch_types=[
            pltpu.VMEM((n_chunks, ch), jnp.int32),
            pltpu.VMEM((ch, Dd), table.dtype),
            pltpu.SemaphoreType.DMA,
        ],
    )
    def k(table_hbm, idx_hbm, out_hbm, idx_v, rows_v, sem):
        wid = lax.axis_index("s") * NC + lax.axis_index("c")
        base = wid * b_per_w
        for c in range(n_chunks):
            pltpu.sync_copy(idx_hbm.at[pl.ds(base + c * ch, ch)], idx_v.at[c])
            pltpu.async_copy(table_hbm.at[idx_v.at[c]], rows_v, sem).wait()
            pltpu.sync_copy(rows_v, out_hbm.at[pl.ds(base + c * ch, ch)])

    return k(table, idx)


def _tc_gmm(xs, w1, w2, w3, block_expert):
    def body(be_ref, x_ref, w1_ref, w3_ref, w2_ref, o_ref):
        x = x_ref[...]
        h = jnp.dot(x, w1_ref[0], preferred_element_type=jnp.float32)
        g = jnp.dot(x, w3_ref[0], preferred_element_type=jnp.float32)
        a = h * jax.nn.sigmoid(h) * g
        o_ref[...] = jnp.dot(a, w2_ref[0], preferred_element_type=jnp.float32)

    grid_spec = pltpu.PrefetchScalarGridSpec(
        num_scalar_prefetch=1,
        grid=(NB,),
        in_specs=[
            pl.BlockSpec((T, D), lambda b, be: (b, 0)),
            pl.BlockSpec((1, D, F), lambda b, be: (be[b], 0, 0)),
            pl.BlockSpec((1, D, F), lambda b, be: (be[b], 0, 0)),
            pl.BlockSpec((1, F, D), lambda b, be: (be[b], 0, 0)),
        ],
        out_specs=pl.BlockSpec((T, D), lambda b, be: (b, 0)),
    )
    return pl.pallas_call(
        body,
        grid_spec=grid_spec,
        out_shape=jax.ShapeDtypeStruct((NP, D), jnp.float32),
    )(block_expert, xs, w1, w3, w2)


def _route(top_ks):
    top_flat = top_ks.reshape(-1).astype(jnp.int32)
    counts = jnp.zeros((E,), jnp.int32).at[top_flat].add(1)
    padded = ((counts + T - 1) // T) * T
    offs_p = jnp.concatenate(
        [jnp.zeros((1,), jnp.int32), jnp.cumsum(padded)[:-1]]
    )
    offs_u = jnp.concatenate(
        [jnp.zeros((1,), jnp.int32), jnp.cumsum(counts)[:-1]]
    )
    order = jnp.argsort(top_flat, stable=True).astype(jnp.int32)
    eid = top_flat[order]
    dest = offs_p[eid] + (jnp.arange(M * K, dtype=jnp.int32) - offs_u[eid])
    sidx = jnp.zeros((NP,), jnp.int32).at[dest].set(order // K)
    pos = jnp.zeros((M * K,), jnp.int32).at[order].set(dest)
    b_idx = jnp.arange(NB, dtype=jnp.int32)
    be = (
        jnp.sum((b_idx[None, :] >= (offs_p // T)[:, None]).astype(jnp.int32), axis=0)
        - 1
    )
    return sidx, pos, be.astype(jnp.int32)


def kernel(hidden_states, top_ks, w1, w2, w3):
    sidx, pos, be = _route(top_ks)
    xs = _sc_gather(hidden_states, sidx, n_chunks=4)
    ys = _tc_gmm(xs, w1, w2, w3, be)
    out = _sc_gather(ys, pos, n_chunks=2)
    return out.reshape(M, K, D)
